# Initial kernel scaffold; baseline (speedup 1.0000x reference)
import jax
import jax.numpy as jnp
from jax.experimental import pallas as pl


def kernel(x, edge_index, W1, b1, W2, b2):
    raise NotImplementedError("write your pallas kernel here")


# speedup vs baseline: 32.1174x; 32.1174x over previous
import functools

import jax
import jax.numpy as jnp
from jax import lax
from jax.experimental import pallas as pl
from jax.experimental.pallas import tpu as pltpu
from jax.experimental.pallas import tpu_sc as plsc

N = 10000
D = 128
H = 128
E = 320000

NUM_CORES = 2
NUM_SUBCORES = 16
NUM_WORKERS = NUM_CORES * NUM_SUBCORES

N_PAD = 10240
RPS = N_PAD // NUM_SUBCORES
EB = 128
NB = 80
NBC = 40
E_PAD = NUM_WORKERS * NB * EB

RT = 400
_G = N // RT


@functools.lru_cache(maxsize=1)
def _sc_kernels():
    mesh = plsc.VectorSubcoreMesh(core_axis_name="c", subcore_axis_name="s")

    @functools.partial(
        pl.kernel,
        mesh=mesh,
        out_type=jax.ShapeDtypeStruct((NUM_WORKERS, N_PAD), jnp.float32),
        compiler_params=pltpu.CompilerParams(needs_layout_passes=False),
        scratch_types=[
            pltpu.VMEM((NB, EB), jnp.int32),
            pltpu.VMEM((N_PAD,), jnp.float32),
            pltpu.SemaphoreType.DMA,
        ],
    )
    def _deg_kernel(dst_hbm, deg_hbm, idx_v, hist_v, dsem):
        cid = lax.axis_index("c")
        sid = lax.axis_index("s")
        wid = cid * NUM_SUBCORES + sid

        idx_cp = pltpu.async_copy(dst_hbm.at[wid], idx_v, dsem)
        z16 = jnp.zeros((16,), jnp.float32)

        def _zero(i, c):
            hist_v[pl.ds(i * 16, 16)] = z16
            return c

        lax.fori_loop(0, N_PAD // 16, _zero, 0)
        idx_cp.wait()

        ones = jnp.ones((16,), jnp.float32)

        def _block(b, c):
            for j in range(EB // 16):
                idx = idx_v[b, pl.ds(j * 16, 16)]
                plsc.addupdate_scatter(hist_v, [idx], ones)
            return c

        lax.fori_loop(0, NB, _block, 0)
        pltpu.sync_copy(hist_v, deg_hbm.at[wid])

    @functools.partial(
        pl.kernel,
        mesh=mesh,
        out_type=jax.ShapeDtypeStruct((NUM_CORES, N, H), jnp.float32),
        scratch_types=[
            pltpu.VMEM((NBC, EB), jnp.int32),
            pltpu.VMEM((NBC, EB), jnp.int32),
            pltpu.VMEM((EB, H), jnp.float32),
            pltpu.VMEM((EB, H), jnp.float32),
            pltpu.VMEM_SHARED((N_PAD, H), jnp.float32),
            pltpu.SemaphoreType.DMA,
            pltpu.SemaphoreType.DMA,
        ],
    )
    def _spmm_kernel(xs_hbm, src_hbm, dst_hbm, p_hbm, idx_s, idx_d, rows0,
                     rows1, acc_sh, sem0, sem1):
        cid = lax.axis_index("c")
        sid = lax.axis_index("s")
        wid = cid * NUM_SUBCORES + sid

        pltpu.sync_copy(src_hbm.at[wid, pl.ds(0, NBC)], idx_s)

        def _gather(b, buf, sem):
            return pltpu.async_copy(xs_hbm.at[idx_s.at[b]], buf, sem)

        _gather(0, rows0, sem0)
        pltpu.sync_copy(dst_hbm.at[wid, pl.ds(0, NBC)], idx_d)

        z16 = jnp.zeros((16,), jnp.float32)

        def _zero_row(r, c):
            for j in range(H // 16):
                rows1[r, pl.ds(j * 16, 16)] = z16
            return c

        lax.fori_loop(0, EB, _zero_row, 0)
        base = sid * RPS
        for k in range(RPS // EB):
            pltpu.sync_copy(rows1, acc_sh.at[pl.ds(base + k * EB, EB)])
        plsc.subcore_barrier()

        def _scatter(b, buf):
            pltpu.sync_copy(buf, acc_sh.at[idx_d.at[b]], add=True)

        def _pair(i, c):
            g = i * 2
            h1 = _gather(g + 1, rows1, sem1)
            pltpu.make_async_copy(xs_hbm.at[idx_s.at[g]], rows0, sem0).wait()
            _scatter(g, rows0)

            @pl.when(g + 2 < NBC)
            def _():
                _gather(g + 2, rows0, sem0)

            h1.wait()
            _scatter(g + 1, rows1)
            return c

        for chunk in range(NB // NBC):
            if chunk > 0:
                pltpu.sync_copy(
                    src_hbm.at[wid, pl.ds(chunk * NBC, NBC)], idx_s
                )
                _gather(0, rows0, sem0)
                pltpu.sync_copy(
                    dst_hbm.at[wid, pl.ds(chunk * NBC, NBC)], idx_d
                )
            lax.fori_loop(0, NBC // 2, _pair, 0)
        plsc.subcore_barrier()

        last = NUM_SUBCORES - 1

        @pl.when(sid < last)
        def _():
            pltpu.sync_copy(
                acc_sh.at[pl.ds(base, RPS)], p_hbm.at[cid, pl.ds(base, RPS)]
            )

        @pl.when(sid == last)
        def _():
            pltpu.sync_copy(
                acc_sh.at[pl.ds(last * RPS, N - last * RPS)],
                p_hbm.at[cid, pl.ds(last * RPS, N - last * RPS)],
            )

    return _deg_kernel, _spmm_kernel


def _tc1_body(x_ref, w1_ref, degp_ref, xs1_ref, dinv_ref):
    dinv = lax.rsqrt(1.0 + jnp.sum(degp_ref[...], axis=0))
    dinv_ref[...] = dinv[:, None]
    xs1_ref[...] = jnp.dot(
        x_ref[...], w1_ref[...], preferred_element_type=jnp.float32
    ) * dinv[:, None]


def _tc2_body(p_ref, xs1_ref, dinv_ref, w2_ref, b1_ref, x1_ref, xs2_ref):
    dinv = dinv_ref[...]
    agg = (p_ref[0] + p_ref[1] + xs1_ref[...]) * dinv + b1_ref[...]
    x1 = jax.nn.sigmoid(agg)
    x1_ref[...] = x1
    xs2_ref[...] = jnp.dot(
        x1, w2_ref[...], preferred_element_type=jnp.float32
    ) * dinv


def _tc3_body(q_ref, xs2_ref, dinv_ref, b2_ref, outbuf_ref, out_ref):
    del outbuf_ref
    dinv = dinv_ref[...]
    y = (q_ref[0] + q_ref[1] + xs2_ref[...]) * dinv + b2_ref[...]
    out_ref[...] = jnp.maximum(y, 0.0)


_tc1 = pl.pallas_call(
    _tc1_body,
    grid=(N_PAD // RPS,),
    in_specs=[
        pl.BlockSpec((RPS, D), lambda i: (i, 0)),
        pl.BlockSpec((D, H), lambda i: (0, 0)),
        pl.BlockSpec((NUM_WORKERS, RPS), lambda i: (0, i)),
    ],
    out_specs=[
        pl.BlockSpec((RPS, H), lambda i: (i, 0)),
        pl.BlockSpec((RPS, 1), lambda i: (i, 0)),
    ],
    out_shape=[
        jax.ShapeDtypeStruct((N_PAD, H), jnp.float32),
        jax.ShapeDtypeStruct((N_PAD, 1), jnp.float32),
    ],
)

_tc2 = pl.pallas_call(
    _tc2_body,
    grid=(_G,),
    in_specs=[
        pl.BlockSpec((NUM_CORES, RT, H), lambda i: (0, i, 0)),
        pl.BlockSpec((RT, H), lambda i: (i, 0)),
        pl.BlockSpec((RT, 1), lambda i: (i, 0)),
        pl.BlockSpec((H, D), lambda i: (0, 0)),
        pl.BlockSpec((1, H), lambda i: (0, 0)),
    ],
    out_specs=[
        pl.BlockSpec((RT, H), lambda i: (i, 0)),
        pl.BlockSpec((RT, D), lambda i: (i, 0)),
    ],
    out_shape=[
        jax.ShapeDtypeStruct((N, H + D), jnp.float32),
        jax.ShapeDtypeStruct((N, D), jnp.float32),
    ],
)

_tc3 = pl.pallas_call(
    _tc3_body,
    grid=(_G,),
    in_specs=[
        pl.BlockSpec((NUM_CORES, RT, D), lambda i: (0, i, 0)),
        pl.BlockSpec((RT, D), lambda i: (i, 0)),
        pl.BlockSpec((RT, 1), lambda i: (i, 0)),
        pl.BlockSpec((1, D), lambda i: (0, 0)),
        pl.BlockSpec(memory_space=pl.ANY),
    ],
    out_specs=pl.BlockSpec((RT, D), lambda i: (i, 1)),
    out_shape=jax.ShapeDtypeStruct((N, H + D), jnp.float32),
    input_output_aliases={4: 0},
)


def kernel(x, edge_index, W1, b1, W2, b2):
    pad = E_PAD - E
    padidx = jnp.arange(pad, dtype=jnp.int32)
    src_r = jnp.concatenate([edge_index[0], padidx % N]).reshape(
        NUM_WORKERS, NB, EB
    )
    dst_r = jnp.concatenate([edge_index[1], N + padidx % (N_PAD - N)]).reshape(
        NUM_WORKERS, NB, EB
    )
    b1r = b1.reshape(1, H)
    b2r = b2.reshape(1, D)
    x_pad = jnp.concatenate([x, jnp.zeros((N_PAD - N, D), jnp.float32)])

    _deg_kernel, _spmm_kernel = _sc_kernels()
    degp = _deg_kernel(dst_r)
    xs1, dinv = _tc1(x_pad, W1, degp)
    p = _spmm_kernel(xs1, src_r, dst_r)
    outbuf, xs2 = _tc2(p, xs1, dinv, W2, b1r)
    q = _spmm_kernel(xs2, src_r, dst_r)
    return _tc3(q, xs2, dinv, b2r, outbuf)


# speedup vs baseline: 32.1282x; 1.0003x over previous
import functools

import jax
import jax.numpy as jnp
from jax import lax
from jax.experimental import pallas as pl
from jax.experimental.pallas import tpu as pltpu
from jax.experimental.pallas import tpu_sc as plsc

N = 10000
D = 128
H = 128
E = 320000

NUM_CORES = 2
NUM_SUBCORES = 16
NUM_WORKERS = NUM_CORES * NUM_SUBCORES

N_PAD = 10240
RPS = N_PAD // NUM_SUBCORES
EB = 128
NB = 80
NBC = 40
E_PAD = NUM_WORKERS * NB * EB

RT = 400
_G = N // RT


@functools.lru_cache(maxsize=1)
def _sc_kernels():
    mesh = plsc.VectorSubcoreMesh(core_axis_name="c", subcore_axis_name="s")

    @functools.partial(
        pl.kernel,
        mesh=mesh,
        out_type=jax.ShapeDtypeStruct((NUM_WORKERS, N_PAD), jnp.float32),
        compiler_params=pltpu.CompilerParams(needs_layout_passes=False),
        scratch_types=[
            pltpu.VMEM((E // NUM_WORKERS,), jnp.int32),
            pltpu.VMEM((N_PAD,), jnp.float32),
            pltpu.SemaphoreType.DMA,
        ],
    )
    def _deg_kernel(ei_hbm, deg_hbm, idx_v, hist_v, dsem):
        cid = lax.axis_index("c")
        sid = lax.axis_index("s")
        wid = cid * NUM_SUBCORES + sid
        epw = E // NUM_WORKERS

        idx_cp = pltpu.async_copy(
            ei_hbm.at[pl.ds(E + wid * epw, epw)], idx_v, dsem
        )
        z16 = jnp.zeros((16,), jnp.float32)

        def _zero(i, c):
            hist_v[pl.ds(i * 16, 16)] = z16
            return c

        lax.fori_loop(0, N_PAD // 16, _zero, 0)
        idx_cp.wait()

        ones = jnp.ones((16,), jnp.float32)

        def _grp(g, c):
            idx = idx_v[pl.ds(g * 16, 16)]
            plsc.addupdate_scatter(hist_v, [idx], ones)
            return c

        lax.fori_loop(0, epw // 16, _grp, 0)
        pltpu.sync_copy(hist_v, deg_hbm.at[wid])

    @functools.partial(
        pl.kernel,
        mesh=mesh,
        out_type=jax.ShapeDtypeStruct((NUM_CORES, N, H), jnp.float32),
        scratch_types=[
            pltpu.VMEM((NBC, EB), jnp.int32),
            pltpu.VMEM((NBC, EB), jnp.int32),
            pltpu.VMEM((EB, H), jnp.float32),
            pltpu.VMEM((EB, H), jnp.float32),
            pltpu.VMEM_SHARED((N_PAD, H), jnp.float32),
            pltpu.SemaphoreType.DMA,
            pltpu.SemaphoreType.DMA,
        ],
    )
    def _spmm_kernel(xs_hbm, src_hbm, dst_hbm, p_hbm, idx_s, idx_d, rows0,
                     rows1, acc_sh, sem0, sem1):
        cid = lax.axis_index("c")
        sid = lax.axis_index("s")
        wid = cid * NUM_SUBCORES + sid

        pltpu.sync_copy(src_hbm.at[wid, pl.ds(0, NBC)], idx_s)

        def _gather(b, buf, sem):
            return pltpu.async_copy(xs_hbm.at[idx_s.at[b]], buf, sem)

        _gather(0, rows0, sem0)
        pltpu.sync_copy(dst_hbm.at[wid, pl.ds(0, NBC)], idx_d)

        z16 = jnp.zeros((16,), jnp.float32)

        def _zero_row(r, c):
            for j in range(H // 16):
                rows1[r, pl.ds(j * 16, 16)] = z16
            return c

        lax.fori_loop(0, EB, _zero_row, 0)
        base = sid * RPS
        for k in range(RPS // EB):
            pltpu.sync_copy(rows1, acc_sh.at[pl.ds(base + k * EB, EB)])
        plsc.subcore_barrier()

        def _scatter(b, buf):
            pltpu.sync_copy(buf, acc_sh.at[idx_d.at[b]], add=True)

        def _pair(i, c):
            g = i * 2
            h1 = _gather(g + 1, rows1, sem1)
            pltpu.make_async_copy(xs_hbm.at[idx_s.at[g]], rows0, sem0).wait()
            _scatter(g, rows0)

            @pl.when(g + 2 < NBC)
            def _():
                _gather(g + 2, rows0, sem0)

            h1.wait()
            _scatter(g + 1, rows1)
            return c

        for chunk in range(NB // NBC):
            if chunk > 0:
                pltpu.sync_copy(
                    src_hbm.at[wid, pl.ds(chunk * NBC, NBC)], idx_s
                )
                _gather(0, rows0, sem0)
                pltpu.sync_copy(
                    dst_hbm.at[wid, pl.ds(chunk * NBC, NBC)], idx_d
                )
            lax.fori_loop(0, NBC // 2, _pair, 0)
        plsc.subcore_barrier()

        last = NUM_SUBCORES - 1

        @pl.when(sid < last)
        def _():
            pltpu.sync_copy(
                acc_sh.at[pl.ds(base, RPS)], p_hbm.at[cid, pl.ds(base, RPS)]
            )

        @pl.when(sid == last)
        def _():
            pltpu.sync_copy(
                acc_sh.at[pl.ds(last * RPS, N - last * RPS)],
                p_hbm.at[cid, pl.ds(last * RPS, N - last * RPS)],
            )

    return _deg_kernel, _spmm_kernel


def _tc1_body(x_ref, w1_ref, degp_ref, xs1_ref, dinv_ref):
    dinv = lax.rsqrt(1.0 + jnp.sum(degp_ref[...], axis=0))
    dinv_ref[...] = dinv[:, None]
    xs1_ref[...] = jnp.dot(
        x_ref[...], w1_ref[...], preferred_element_type=jnp.float32
    ) * dinv[:, None]


def _tc2_body(p_ref, xs1_ref, dinv_ref, w2_ref, b1_ref, x1_ref, xs2_ref):
    dinv = dinv_ref[...]
    agg = (p_ref[0] + p_ref[1] + xs1_ref[...]) * dinv + b1_ref[...]
    x1 = jax.nn.sigmoid(agg)
    x1_ref[...] = x1
    xs2_ref[...] = jnp.dot(
        x1, w2_ref[...], preferred_element_type=jnp.float32
    ) * dinv


def _tc3_body(q_ref, xs2_ref, dinv_ref, b2_ref, outbuf_ref, out_ref):
    del outbuf_ref
    dinv = dinv_ref[...]
    y = (q_ref[0] + q_ref[1] + xs2_ref[...]) * dinv + b2_ref[...]
    out_ref[...] = jnp.maximum(y, 0.0)


_tc1 = pl.pallas_call(
    _tc1_body,
    grid=(N_PAD // RPS,),
    in_specs=[
        pl.BlockSpec((RPS, D), lambda i: (i, 0)),
        pl.BlockSpec((D, H), lambda i: (0, 0)),
        pl.BlockSpec((NUM_WORKERS, RPS), lambda i: (0, i)),
    ],
    out_specs=[
        pl.BlockSpec((RPS, H), lambda i: (i, 0)),
        pl.BlockSpec((RPS, 1), lambda i: (i, 0)),
    ],
    out_shape=[
        jax.ShapeDtypeStruct((N_PAD, H), jnp.float32),
        jax.ShapeDtypeStruct((N_PAD, 1), jnp.float32),
    ],
)

_tc2 = pl.pallas_call(
    _tc2_body,
    grid=(_G,),
    in_specs=[
        pl.BlockSpec((NUM_CORES, RT, H), lambda i: (0, i, 0)),
        pl.BlockSpec((RT, H), lambda i: (i, 0)),
        pl.BlockSpec((RT, 1), lambda i: (i, 0)),
        pl.BlockSpec((H, D), lambda i: (0, 0)),
        pl.BlockSpec((1, H), lambda i: (0, 0)),
    ],
    out_specs=[
        pl.BlockSpec((RT, H), lambda i: (i, 0)),
        pl.BlockSpec((RT, D), lambda i: (i, 0)),
    ],
    out_shape=[
        jax.ShapeDtypeStruct((N, H + D), jnp.float32),
        jax.ShapeDtypeStruct((N, D), jnp.float32),
    ],
)

_tc3 = pl.pallas_call(
    _tc3_body,
    grid=(_G,),
    in_specs=[
        pl.BlockSpec((NUM_CORES, RT, D), lambda i: (0, i, 0)),
        pl.BlockSpec((RT, D), lambda i: (i, 0)),
        pl.BlockSpec((RT, 1), lambda i: (i, 0)),
        pl.BlockSpec((1, D), lambda i: (0, 0)),
        pl.BlockSpec(memory_space=pl.ANY),
    ],
    out_specs=pl.BlockSpec((RT, D), lambda i: (i, 1)),
    out_shape=jax.ShapeDtypeStruct((N, H + D), jnp.float32),
    input_output_aliases={4: 0},
)


def kernel(x, edge_index, W1, b1, W2, b2):
    pad = E_PAD - E
    padidx = jnp.arange(pad, dtype=jnp.int32)
    src_r = jnp.concatenate([edge_index[0], padidx % N]).reshape(
        NUM_WORKERS, NB, EB
    )
    dst_r = jnp.concatenate([edge_index[1], N + padidx % (N_PAD - N)]).reshape(
        NUM_WORKERS, NB, EB
    )
    b1r = b1.reshape(1, H)
    b2r = b2.reshape(1, D)
    x_pad = jnp.concatenate([x, jnp.zeros((N_PAD - N, D), jnp.float32)])

    _deg_kernel, _spmm_kernel = _sc_kernels()
    degp = _deg_kernel(edge_index.reshape(2 * E))
    xs1, dinv = _tc1(x_pad, W1, degp)
    p = _spmm_kernel(xs1, src_r, dst_r)
    outbuf, xs2 = _tc2(p, xs1, dinv, W2, b1r)
    q = _spmm_kernel(xs2, src_r, dst_r)
    return _tc3(q, xs2, dinv, b2r, outbuf)


# speedup vs baseline: 32.7963x; 1.0208x over previous
import functools

import jax
import jax.numpy as jnp
from jax import lax
from jax.experimental import pallas as pl
from jax.experimental.pallas import tpu as pltpu
from jax.experimental.pallas import tpu_sc as plsc

N = 10000
D = 128
H = 128
E = 320000

NUM_CORES = 2
NUM_SUBCORES = 16
NUM_WORKERS = NUM_CORES * NUM_SUBCORES

N_PAD = 10240
RPS = N_PAD // NUM_SUBCORES
EB = 128
NB = 80
NBC = 40
CH = NBC * EB
EPW = E // NUM_WORKERS
PADW = NB * EB - EPW
DMAX = EB - EPW % EB
DEGL = EPW + DMAX
CHL = (CH + EB, CH - PADW + DMAX)

RT = 400
_G = N // RT


@functools.lru_cache(maxsize=1)
def _sc_kernels():
    mesh = plsc.VectorSubcoreMesh(core_axis_name="c", subcore_axis_name="s")

    @functools.partial(
        pl.kernel,
        mesh=mesh,
        out_type=jax.ShapeDtypeStruct((NUM_WORKERS, N_PAD), jnp.float32),
        compiler_params=pltpu.CompilerParams(needs_layout_passes=False),
        scratch_types=[
            pltpu.VMEM((EPW,), jnp.int32),
            pltpu.VMEM((N_PAD,), jnp.float32),
            pltpu.SemaphoreType.DMA,
        ],
    )
    def _deg_kernel(ei_hbm, deg_hbm, idx_v, hist_v, dsem):
        cid = lax.axis_index("c")
        sid = lax.axis_index("s")
        wid = cid * NUM_SUBCORES + sid

        idx_cp = pltpu.async_copy(
            ei_hbm.at[pl.ds(E + wid * EPW, EPW)], idx_v, dsem
        )
        z16 = jnp.zeros((16,), jnp.float32)

        def _zero(i, c):
            hist_v[pl.ds(i * 16, 16)] = z16
            return c

        lax.fori_loop(0, N_PAD // 16, _zero, 0)
        idx_cp.wait()

        ones = jnp.ones((16,), jnp.float32)

        def _grp(g, c):
            idx = idx_v[pl.ds(g * 16, 16)]
            plsc.addupdate_scatter(hist_v, [idx], ones)
            return c

        lax.fori_loop(0, EPW // 16, _grp, 0)
        pltpu.sync_copy(hist_v, deg_hbm.at[wid])

    @functools.partial(
        pl.kernel,
        mesh=mesh,
        out_type=jax.ShapeDtypeStruct((NUM_CORES, N, H), jnp.float32),
        scratch_types=[
            pltpu.VMEM((CH,), jnp.int32),
            pltpu.VMEM((CH,), jnp.int32),
            pltpu.VMEM((NBC, EB), jnp.int32),
            pltpu.VMEM((EB, H), jnp.float32),
            pltpu.VMEM((EB, H), jnp.float32),
            pltpu.VMEM_SHARED((N_PAD, H), jnp.float32),
            pltpu.SemaphoreType.DMA,
            pltpu.SemaphoreType.DMA,
        ],
    )
    def _spmm_kernel(xs_hbm, ei_hbm, p_hbm, src1d, dst1d, dst2d, rows0,
                     rows1, acc_sh, sem0, sem1):
        cid = lax.axis_index("c")
        sid = lax.axis_index("s")
        wid = cid * NUM_SUBCORES + sid

        def _load_chunk(chunk):
            nreal = CH if chunk == 0 else CH - PADW
            off = wid * EPW + chunk * CH
            pltpu.sync_copy(
                ei_hbm.at[pl.ds(off, nreal)], src1d.at[pl.ds(0, nreal)]
            )
            pltpu.sync_copy(
                ei_hbm.at[pl.ds(E + off, nreal)], dst1d.at[pl.ds(0, nreal)]
            )
            if chunk != 0:
                lane = lax.broadcasted_iota(jnp.int32, (16,), 0)
                for k in range(PADW // 16):
                    v = lane + (k * 16)
                    src1d[pl.ds(nreal + k * 16, 16)] = v
                    dst1d[pl.ds(nreal + k * 16, 16)] = N + v

        def _repack_dst(b, c):
            for j in range(EB // 16):
                dst2d[b, pl.ds(j * 16, 16)] = dst1d[pl.ds(b * EB + j * 16, 16)]
            return c

        _load_chunk(0)

        def _gather(b, buf, sem):
            return pltpu.async_copy(
                xs_hbm.at[src1d.at[pl.ds(b * EB, EB)]], buf, sem
            )

        _gather(0, rows0, sem0)
        lax.fori_loop(0, NBC, _repack_dst, 0)

        z16 = jnp.zeros((16,), jnp.float32)

        def _zero_row(r, c):
            for j in range(H // 16):
                rows1[r, pl.ds(j * 16, 16)] = z16
            return c

        lax.fori_loop(0, EB, _zero_row, 0)
        base = sid * RPS
        for k in range(RPS // EB):
            pltpu.sync_copy(rows1, acc_sh.at[pl.ds(base + k * EB, EB)])
        plsc.subcore_barrier()

        def _scatter(b, buf):
            pltpu.sync_copy(buf, acc_sh.at[dst2d.at[b]], add=True)

        def _pair(i, c):
            g = i * 2
            h1 = _gather(g + 1, rows1, sem1)
            pltpu.make_async_copy(
                xs_hbm.at[src1d.at[pl.ds(g * EB, EB)]], rows0, sem0
            ).wait()
            _scatter(g, rows0)

            @pl.when(g + 2 < NBC)
            def _():
                _gather(g + 2, rows0, sem0)

            h1.wait()
            _scatter(g + 1, rows1)
            return c

        for chunk in range(NB // NBC):
            if chunk > 0:
                _load_chunk(chunk)
                _gather(0, rows0, sem0)
                lax.fori_loop(0, NBC, _repack_dst, 0)
            lax.fori_loop(0, NBC // 2, _pair, 0)
        plsc.subcore_barrier()

        last = NUM_SUBCORES - 1

        @pl.when(sid < last)
        def _():
            pltpu.sync_copy(
                acc_sh.at[pl.ds(base, RPS)], p_hbm.at[cid, pl.ds(base, RPS)]
            )

        @pl.when(sid == last)
        def _():
            pltpu.sync_copy(
                acc_sh.at[pl.ds(last * RPS, N - last * RPS)],
                p_hbm.at[cid, pl.ds(last * RPS, N - last * RPS)],
            )

    return _deg_kernel, _spmm_kernel


def _tc1_body(x_ref, w1_ref, degp_ref, xs1_ref, dinv_ref):
    dinv = lax.rsqrt(1.0 + jnp.sum(degp_ref[...], axis=0))
    dinv_ref[...] = dinv[:, None]
    xs1_ref[...] = jnp.dot(
        x_ref[...], w1_ref[...], preferred_element_type=jnp.float32
    ) * dinv[:, None]


def _tc2_body(p_ref, xs1_ref, dinv_ref, w2_ref, b1_ref, x1_ref, xs2_ref):
    dinv = dinv_ref[...]
    agg = (p_ref[0] + p_ref[1] + xs1_ref[...]) * dinv + b1_ref[...]
    x1 = jax.nn.sigmoid(agg)
    x1_ref[...] = x1
    xs2_ref[...] = jnp.dot(
        x1, w2_ref[...], preferred_element_type=jnp.float32
    ) * dinv


def _tc3_body(q_ref, xs2_ref, dinv_ref, b2_ref, outbuf_ref, out_ref):
    del outbuf_ref
    dinv = dinv_ref[...]
    y = (q_ref[0] + q_ref[1] + xs2_ref[...]) * dinv + b2_ref[...]
    out_ref[...] = jnp.maximum(y, 0.0)


_tc1 = pl.pallas_call(
    _tc1_body,
    grid=(N_PAD // RPS,),
    in_specs=[
        pl.BlockSpec((RPS, D), lambda i: (i, 0)),
        pl.BlockSpec((D, H), lambda i: (0, 0)),
        pl.BlockSpec((NUM_WORKERS, RPS), lambda i: (0, i)),
    ],
    out_specs=[
        pl.BlockSpec((RPS, H), lambda i: (i, 0)),
        pl.BlockSpec((RPS, 1), lambda i: (i, 0)),
    ],
    out_shape=[
        jax.ShapeDtypeStruct((N_PAD, H), jnp.float32),
        jax.ShapeDtypeStruct((N_PAD, 1), jnp.float32),
    ],
)

_tc2 = pl.pallas_call(
    _tc2_body,
    grid=(_G,),
    in_specs=[
        pl.BlockSpec((NUM_CORES, RT, H), lambda i: (0, i, 0)),
        pl.BlockSpec((RT, H), lambda i: (i, 0)),
        pl.BlockSpec((RT, 1), lambda i: (i, 0)),
        pl.BlockSpec((H, D), lambda i: (0, 0)),
        pl.BlockSpec((1, H), lambda i: (0, 0)),
    ],
    out_specs=[
        pl.BlockSpec((RT, H), lambda i: (i, 0)),
        pl.BlockSpec((RT, D), lambda i: (i, 0)),
    ],
    out_shape=[
        jax.ShapeDtypeStruct((N, H + D), jnp.float32),
        jax.ShapeDtypeStruct((N, D), jnp.float32),
    ],
)

_tc3 = pl.pallas_call(
    _tc3_body,
    grid=(_G,),
    in_specs=[
        pl.BlockSpec((NUM_CORES, RT, D), lambda i: (0, i, 0)),
        pl.BlockSpec((RT, D), lambda i: (i, 0)),
        pl.BlockSpec((RT, 1), lambda i: (i, 0)),
        pl.BlockSpec((1, D), lambda i: (0, 0)),
        pl.BlockSpec(memory_space=pl.ANY),
    ],
    out_specs=pl.BlockSpec((RT, D), lambda i: (i, 1)),
    out_shape=jax.ShapeDtypeStruct((N, H + D), jnp.float32),
    input_output_aliases={4: 0},
)


def kernel(x, edge_index, W1, b1, W2, b2):
    b1r = b1.reshape(1, H)
    b2r = b2.reshape(1, D)
    x_pad = jnp.concatenate([x, jnp.zeros((N_PAD - N, D), jnp.float32)])

    _deg_kernel, _spmm_kernel = _sc_kernels()
    ei_flat = edge_index.reshape(2 * E)
    degp = _deg_kernel(ei_flat)
    xs1, dinv = _tc1(x_pad, W1, degp)
    p = _spmm_kernel(xs1, ei_flat)
    outbuf, xs2 = _tc2(p, xs1, dinv, W2, b1r)
    q = _spmm_kernel(xs2, ei_flat)
    return _tc3(q, xs2, dinv, b2r, outbuf)


# speedup vs baseline: 33.2612x; 1.0142x over previous
import functools

import jax
import jax.numpy as jnp
from jax import lax
from jax.experimental import pallas as pl
from jax.experimental.pallas import tpu as pltpu
from jax.experimental.pallas import tpu_sc as plsc

N = 10000
D = 128
H = 128
E = 320000

NUM_CORES = 2
NUM_SUBCORES = 16
NUM_WORKERS = NUM_CORES * NUM_SUBCORES

N_PAD = 10240
RPS = N_PAD // NUM_SUBCORES
EB = 128
NB = 80
NBC = 40
CH = NBC * EB
EPW = E // NUM_WORKERS
PADW = NB * EB - EPW
DMAX = EB - EPW % EB
DEGL = EPW + DMAX
CHL = (CH + EB, CH - PADW + DMAX)

RT = 400
_G = N // RT


@functools.lru_cache(maxsize=1)
def _sc_kernels():
    mesh = plsc.VectorSubcoreMesh(core_axis_name="c", subcore_axis_name="s")

    @functools.partial(
        pl.kernel,
        mesh=mesh,
        out_type=jax.ShapeDtypeStruct((NUM_WORKERS, N_PAD), jnp.float32),
        compiler_params=pltpu.CompilerParams(needs_layout_passes=False),
        scratch_types=[
            pltpu.VMEM((1, 2, DEGL), jnp.int32),
            pltpu.VMEM((N_PAD,), jnp.float32),
            pltpu.SemaphoreType.DMA,
        ],
    )
    def _deg_kernel(ei_hbm, deg_hbm, idx_v, hist_v, dsem):
        cid = lax.axis_index("c")
        sid = lax.axis_index("s")
        wid = cid * NUM_SUBCORES + sid
        delta = lax.rem(wid * (EPW % EB), EB)
        astart = pl.multiple_of(wid * EPW - delta, EB)
        idx_cp = pltpu.async_copy(
            ei_hbm.at[:, pl.ds(astart, DEGL)], idx_v.at[0], dsem
        )
        z16 = jnp.zeros((16,), jnp.float32)

        def _zero(i, c):
            hist_v[pl.ds(i * 16, 16)] = z16
            return c

        lax.fori_loop(0, N_PAD // 16, _zero, 0)
        idx_cp.wait()

        ones = jnp.ones((16,), jnp.float32)

        def _grp(g, c):
            idx = idx_v[0, 1, pl.ds(delta + g * 16, 16)]
            plsc.addupdate_scatter(hist_v, [idx], ones)
            return c

        lax.fori_loop(0, EPW // 16, _grp, 0)
        pltpu.sync_copy(hist_v, deg_hbm.at[wid])

    @functools.partial(
        pl.kernel,
        mesh=mesh,
        out_type=jax.ShapeDtypeStruct((NUM_CORES, N, H), jnp.float32),
        scratch_types=[
            pltpu.VMEM((CH,), jnp.int32),
            pltpu.VMEM((CH,), jnp.int32),
            pltpu.VMEM((NBC, EB), jnp.int32),
            pltpu.VMEM((EB, H), jnp.float32),
            pltpu.VMEM((EB, H), jnp.float32),
            pltpu.VMEM_SHARED((N_PAD, H), jnp.float32),
            pltpu.SemaphoreType.DMA,
            pltpu.SemaphoreType.DMA,
        ],
    )
    def _spmm_kernel(xs_hbm, ei_hbm, p_hbm, src1d, dst1d, dst2d, rows0,
                     rows1, acc_sh, sem0, sem1):
        cid = lax.axis_index("c")
        sid = lax.axis_index("s")
        wid = cid * NUM_SUBCORES + sid

        def _load_chunk(chunk):
            nreal = CH if chunk == 0 else CH - PADW
            off = wid * EPW + chunk * CH
            pltpu.sync_copy(
                ei_hbm.at[pl.ds(off, nreal)], src1d.at[pl.ds(0, nreal)]
            )
            pltpu.sync_copy(
                ei_hbm.at[pl.ds(E + off, nreal)], dst1d.at[pl.ds(0, nreal)]
            )
            if chunk != 0:
                lane = lax.broadcasted_iota(jnp.int32, (16,), 0)
                for k in range(PADW // 16):
                    v = lane + (k * 16)
                    src1d[pl.ds(nreal + k * 16, 16)] = v
                    dst1d[pl.ds(nreal + k * 16, 16)] = N + v

        def _repack_dst(b, c):
            for j in range(EB // 16):
                dst2d[b, pl.ds(j * 16, 16)] = dst1d[pl.ds(b * EB + j * 16, 16)]
            return c

        _load_chunk(0)

        def _gather(b, buf, sem):
            return pltpu.async_copy(
                xs_hbm.at[src1d.at[pl.ds(b * EB, EB)]], buf, sem
            )

        _gather(0, rows0, sem0)
        lax.fori_loop(0, NBC, _repack_dst, 0)

        z16 = jnp.zeros((16,), jnp.float32)

        def _zero_row(r, c):
            for j in range(H // 16):
                rows1[r, pl.ds(j * 16, 16)] = z16
            return c

        lax.fori_loop(0, EB, _zero_row, 0)
        base = sid * RPS
        for k in range(RPS // EB):
            pltpu.sync_copy(rows1, acc_sh.at[pl.ds(base + k * EB, EB)])
        plsc.subcore_barrier()

        def _scatter(b, buf):
            pltpu.sync_copy(buf, acc_sh.at[dst2d.at[b]], add=True)

        def _pair(i, c):
            g = i * 2
            h1 = _gather(g + 1, rows1, sem1)
            pltpu.make_async_copy(
                xs_hbm.at[src1d.at[pl.ds(g * EB, EB)]], rows0, sem0
            ).wait()
            _scatter(g, rows0)

            @pl.when(g + 2 < NBC)
            def _():
                _gather(g + 2, rows0, sem0)

            h1.wait()
            _scatter(g + 1, rows1)
            return c

        for chunk in range(NB // NBC):
            if chunk > 0:
                _load_chunk(chunk)
                _gather(0, rows0, sem0)
                lax.fori_loop(0, NBC, _repack_dst, 0)
            lax.fori_loop(0, NBC // 2, _pair, 0)
        plsc.subcore_barrier()

        last = NUM_SUBCORES - 1

        @pl.when(sid < last)
        def _():
            pltpu.sync_copy(
                acc_sh.at[pl.ds(base, RPS)], p_hbm.at[cid, pl.ds(base, RPS)]
            )

        @pl.when(sid == last)
        def _():
            pltpu.sync_copy(
                acc_sh.at[pl.ds(last * RPS, N - last * RPS)],
                p_hbm.at[cid, pl.ds(last * RPS, N - last * RPS)],
            )

    return _deg_kernel, _spmm_kernel


def _tc1_body(x_ref, w1_ref, degp_ref, xs1_ref, dinv_ref):
    dinv = lax.rsqrt(1.0 + jnp.sum(degp_ref[...], axis=0))
    dinv_ref[...] = dinv[:, None]
    xs1_ref[...] = jnp.dot(
        x_ref[...], w1_ref[...], preferred_element_type=jnp.float32
    ) * dinv[:, None]


def _tc2_body(p_ref, xs1_ref, dinv_ref, w2_ref, b1_ref, x1_ref, xs2_ref):
    dinv = dinv_ref[...]
    agg = (p_ref[0] + p_ref[1] + xs1_ref[...]) * dinv + b1_ref[...]
    x1 = jax.nn.sigmoid(agg)
    x1_ref[...] = x1
    xs2_ref[...] = jnp.dot(
        x1, w2_ref[...], preferred_element_type=jnp.float32
    ) * dinv


def _tc3_body(q_ref, xs2_ref, dinv_ref, b2_ref, outbuf_ref, out_ref):
    del outbuf_ref
    dinv = dinv_ref[...]
    y = (q_ref[0] + q_ref[1] + xs2_ref[...]) * dinv + b2_ref[...]
    out_ref[...] = jnp.maximum(y, 0.0)


_tc1 = pl.pallas_call(
    _tc1_body,
    grid=(N_PAD // RPS,),
    in_specs=[
        pl.BlockSpec((RPS, D), lambda i: (i, 0)),
        pl.BlockSpec((D, H), lambda i: (0, 0)),
        pl.BlockSpec((NUM_WORKERS, RPS), lambda i: (0, i)),
    ],
    out_specs=[
        pl.BlockSpec((RPS, H), lambda i: (i, 0)),
        pl.BlockSpec((RPS, 1), lambda i: (i, 0)),
    ],
    out_shape=[
        jax.ShapeDtypeStruct((N_PAD, H), jnp.float32),
        jax.ShapeDtypeStruct((N_PAD, 1), jnp.float32),
    ],
)

_tc2 = pl.pallas_call(
    _tc2_body,
    grid=(_G,),
    in_specs=[
        pl.BlockSpec((NUM_CORES, RT, H), lambda i: (0, i, 0)),
        pl.BlockSpec((RT, H), lambda i: (i, 0)),
        pl.BlockSpec((RT, 1), lambda i: (i, 0)),
        pl.BlockSpec((H, D), lambda i: (0, 0)),
        pl.BlockSpec((1, H), lambda i: (0, 0)),
    ],
    out_specs=[
        pl.BlockSpec((RT, H), lambda i: (i, 0)),
        pl.BlockSpec((RT, D), lambda i: (i, 0)),
    ],
    out_shape=[
        jax.ShapeDtypeStruct((N, H + D), jnp.float32),
        jax.ShapeDtypeStruct((N, D), jnp.float32),
    ],
)

_tc3 = pl.pallas_call(
    _tc3_body,
    grid=(_G,),
    in_specs=[
        pl.BlockSpec((NUM_CORES, RT, D), lambda i: (0, i, 0)),
        pl.BlockSpec((RT, D), lambda i: (i, 0)),
        pl.BlockSpec((RT, 1), lambda i: (i, 0)),
        pl.BlockSpec((1, D), lambda i: (0, 0)),
        pl.BlockSpec(memory_space=pl.ANY),
    ],
    out_specs=pl.BlockSpec((RT, D), lambda i: (i, 1)),
    out_shape=jax.ShapeDtypeStruct((N, H + D), jnp.float32),
    input_output_aliases={4: 0},
)


def kernel(x, edge_index, W1, b1, W2, b2):
    b1r = b1.reshape(1, H)
    b2r = b2.reshape(1, D)
    x_pad = jnp.concatenate([x, jnp.zeros((N_PAD - N, D), jnp.float32)])

    _deg_kernel, _spmm_kernel = _sc_kernels()
    ei_flat = edge_index.reshape(2 * E)
    degp = _deg_kernel(edge_index)
    xs1, dinv = _tc1(x_pad, W1, degp)
    p = _spmm_kernel(xs1, ei_flat)
    outbuf, xs2 = _tc2(p, xs1, dinv, W2, b1r)
    q = _spmm_kernel(xs2, ei_flat)
    return _tc3(q, xs2, dinv, b2r, outbuf)
